# baseline (speedup 1.0000x reference)
import functools

import jax
import jax.numpy as jnp
from jax import lax
from jax.experimental import pallas as pl
from jax.experimental.pallas import tpu as pltpu
from jax.experimental.pallas import tpu_sc as plsc

NUM_EXPERTS = 64
TOP_K = 8
E_BLK = 2
SPLIT = 4

SC_CORES = 2
SC_SUBCORES = 16
SC_LANES = 16
_NEG_BIG = -3.4e38


def _logits_body(hs_ref, gw_ref, out_ref):
    out_ref[...] = jax.lax.dot_general(
        hs_ref[...], gw_ref[...], (((1,), (1,)), ((), ())),
        preferred_element_type=jnp.float32)


def _vshuffle(x, idx):
    return x.at[idx].get(mode="promise_in_bounds")


def _all_lanes(x, op):
    lane = lax.broadcasted_iota(jnp.int32, (SC_LANES,), 0)
    for sh in (8, 4, 2, 1):
        x = op(x, _vshuffle(x, jnp.bitwise_xor(lane, sh)))
    return x


def _router_sc_body(logits_hbm, comb_hbm, row_v, comb_v):
    T = logits_hbm.shape[0]
    E = logits_hbm.shape[1]
    nvec = E // SC_LANES
    n_workers = SC_CORES * SC_SUBCORES
    per_worker = T // n_workers
    wid = lax.axis_index("s") * SC_CORES + lax.axis_index("c")
    gidx = [
        lax.broadcasted_iota(jnp.int32, (SC_LANES,), 0) + v * SC_LANES
        for v in range(nvec)
    ]
    for r in range(per_worker):
        t = wid * per_worker + r
        pltpu.sync_copy(logits_hbm.at[t], row_v)
        orig = [row_v[pl.ds(v * SC_LANES, SC_LANES)] for v in range(nvec)]
        work = list(orig)
        mask = [jnp.zeros((SC_LANES,), jnp.float32) for _ in range(nvec)]
        row_max = None
        for _ in range(TOP_K):
            m = work[0]
            for v in range(1, nvec):
                m = jnp.maximum(m, work[v])
            mx = _all_lanes(m, jnp.maximum)
            if row_max is None:
                row_max = mx
            cand = [jnp.where(work[v] == mx, gidx[v], E) for v in range(nvec)]
            cm = cand[0]
            for v in range(1, nvec):
                cm = jnp.minimum(cm, cand[v])
            jstar = _all_lanes(cm, jnp.minimum)
            for v in range(nvec):
                sel = gidx[v] == jstar
                mask[v] = jnp.where(sel, 1.0, mask[v])
                work[v] = jnp.where(sel, _NEG_BIG, work[v])
        p = [jnp.exp(orig[v] - row_max) * mask[v] for v in range(nvec)]
        s = p[0]
        for v in range(1, nvec):
            s = s + p[v]
        denom = _all_lanes(s, jnp.add)
        for v in range(nvec):
            comb_v[pl.ds(v * SC_LANES, SC_LANES)] = p[v] / denom
        pltpu.sync_copy(comb_v, comb_hbm.at[t])


def _gateup_body(*refs):
    hs_ref = refs[0]
    gp_refs = refs[1:1 + SPLIT]
    up_refs = refs[1 + SPLIT:1 + 2 * SPLIT]
    act_ref = refs[1 + 2 * SPLIT]

    hs = hs_ref[...]
    T, H = hs.shape
    Ip = gp_refs[0].shape[1]
    I = Ip * SPLIT

    for p in range(SPLIT):
        gp = gp_refs[p][...].reshape(E_BLK * Ip, H)
        up = up_refs[p][...].reshape(E_BLK * Ip, H)
        g = jax.lax.dot_general(hs, gp, (((1,), (1,)), ((), ())),
                                preferred_element_type=jnp.float32)
        u = jax.lax.dot_general(hs, up, (((1,), (1,)), ((), ())),
                                preferred_element_type=jnp.float32)
        a = g * jax.nn.sigmoid(g) * u
        for j in range(E_BLK):
            act_ref[:, j * I + p * Ip:j * I + (p + 1) * Ip] = (
                a[:, j * Ip:(j + 1) * Ip])


def _down_body(*refs):
    act_ref, comb_ref = refs[0], refs[1]
    dp_refs = refs[2:2 + SPLIT]
    out_ref = refs[2 + SPLIT]

    i = pl.program_id(0)
    T = act_ref.shape[0]
    I = act_ref.shape[1] // E_BLK
    Hp = dp_refs[0].shape[1]

    @pl.when(i == 0)
    def _init():
        out_ref[...] = jnp.zeros_like(out_ref)

    E = comb_ref.shape[1]
    colid = jax.lax.broadcasted_iota(jnp.int32, (T, E), 1)
    comb = comb_ref[...]
    aw = []
    for j in range(E_BLK):
        e = i * E_BLK + j
        w = jnp.sum(jnp.where(colid == e, comb, 0.0), axis=1,
                    keepdims=True)
        aw.append(act_ref[:, j * I:(j + 1) * I] * w)

    for p in range(SPLIT):
        acc = out_ref[:, p * Hp:(p + 1) * Hp]
        for j in range(E_BLK):
            acc = acc + jax.lax.dot_general(
                aw[j], dp_refs[p][j], (((1,), (1,)), ((), ())),
                preferred_element_type=jnp.float32)
        out_ref[:, p * Hp:(p + 1) * Hp] = acc


@functools.partial(jax.jit, static_argnames=())
def kernel(hidden_states, gate_w, gate_proj, up_proj, down_proj):
    B, S, H = hidden_states.shape
    T = B * S
    hs = hidden_states.reshape(T, H)
    E = gate_w.shape[0]
    I = gate_proj.shape[1]
    Ip = I // SPLIT
    Hp = H // SPLIT

    logits = pl.pallas_call(
        _logits_body,
        out_shape=jax.ShapeDtypeStruct((T, E), jnp.float32),
    )(hs, gate_w)

    comb = pl.kernel(
        _router_sc_body,
        out_type=jax.ShapeDtypeStruct((T, E), jnp.float32),
        mesh=plsc.VectorSubcoreMesh(core_axis_name="c", subcore_axis_name="s"),
        scratch_types=[
            pltpu.VMEM((E,), jnp.float32),
            pltpu.VMEM((E,), jnp.float32),
        ],
    )(logits)

    gu_specs = [pl.BlockSpec((T, H), lambda i: (0, 0))]
    for p in range(SPLIT):
        gu_specs.append(
            pl.BlockSpec((E_BLK, Ip, H), lambda i, p=p: (i, p, 0)))
    for p in range(SPLIT):
        gu_specs.append(
            pl.BlockSpec((E_BLK, Ip, H), lambda i, p=p: (i, p, 0)))

    act = pl.pallas_call(
        _gateup_body,
        grid=(E // E_BLK,),
        in_specs=gu_specs,
        out_specs=pl.BlockSpec((T, E_BLK * I), lambda i: (0, i)),
        out_shape=jax.ShapeDtypeStruct((T, E * I), jnp.float32),
        compiler_params=pltpu.CompilerParams(
            dimension_semantics=("arbitrary",),
        ),
    )(hs, *([gate_proj] * SPLIT), *([up_proj] * SPLIT))

    dn_specs = [
        pl.BlockSpec((T, E_BLK * I), lambda i: (0, i)),
        pl.BlockSpec((T, E), lambda i: (0, 0)),
    ]
    for p in range(SPLIT):
        dn_specs.append(
            pl.BlockSpec((E_BLK, Hp, I), lambda i, p=p: (i, p, 0)))

    final = pl.pallas_call(
        _down_body,
        grid=(E // E_BLK,),
        in_specs=dn_specs,
        out_specs=pl.BlockSpec((T, H), lambda i: (0, 0)),
        out_shape=jax.ShapeDtypeStruct((T, H), jnp.float32),
        compiler_params=pltpu.CompilerParams(
            dimension_semantics=("arbitrary",),
        ),
    )(act, comb, *([down_proj] * SPLIT))

    return final.reshape(B, S, H), logits


# speedup vs baseline: 1.2093x; 1.2093x over previous
import functools

import jax
import jax.numpy as jnp
from jax.experimental import pallas as pl
from jax.experimental.pallas import tpu as pltpu

NUM_EXPERTS = 64
TOP_K = 8
E_BLK = 2
SPLIT = 8


def _moe_body(*refs):
    hs_ref, gw_ref = refs[0], refs[1]
    gp_refs = refs[2:2 + SPLIT]
    up_refs = refs[2 + SPLIT:2 + 2 * SPLIT]
    dp_refs = refs[2 + 2 * SPLIT:2 + 3 * SPLIT]
    out_ref, logits_ref, comb_ref = refs[2 + 3 * SPLIT:]

    i = pl.program_id(0)
    hs = hs_ref[...]
    T, H = hs.shape
    Ip = gp_refs[0].shape[1]
    I = Ip * SPLIT
    Hp = dp_refs[0].shape[1]

    @pl.when(i == 0)
    def _router():
        logits = jax.lax.dot_general(
            hs, gw_ref[...], (((1,), (1,)), ((), ())),
            preferred_element_type=jnp.float32)
        logits_ref[...] = logits
        probs = jax.nn.softmax(logits, axis=1)
        E = probs.shape[1]
        colid = jax.lax.broadcasted_iota(jnp.int32, (T, E), 1)
        comb = jnp.zeros_like(probs)
        p = probs
        for _ in range(TOP_K):
            m = jnp.max(p, axis=1, keepdims=True)
            idx = jnp.where(p == m, colid, E)
            sel = colid == jnp.min(idx, axis=1, keepdims=True)
            comb = jnp.where(sel, p, comb)
            p = jnp.where(sel, -1.0, p)
        comb = comb / jnp.sum(comb, axis=1, keepdims=True)
        comb_ref[...] = comb
        out_ref[...] = jnp.zeros_like(out_ref)

    a_parts = [None] * (E_BLK * SPLIT)
    for p in range(SPLIT):
        gp = gp_refs[p][...].reshape(E_BLK * Ip, H)
        up = up_refs[p][...].reshape(E_BLK * Ip, H)
        g = jax.lax.dot_general(hs, gp, (((1,), (1,)), ((), ())),
                                preferred_element_type=jnp.float32)
        u = jax.lax.dot_general(hs, up, (((1,), (1,)), ((), ())),
                                preferred_element_type=jnp.float32)
        a = g * jax.nn.sigmoid(g) * u
        for j in range(E_BLK):
            a_parts[j * SPLIT + p] = a[:, j * Ip:(j + 1) * Ip]

    E = comb_ref.shape[1]
    colid = jax.lax.broadcasted_iota(jnp.int32, (T, E), 1)
    comb = comb_ref[...]
    aw = []
    for j in range(E_BLK):
        e = i * E_BLK + j
        w = jnp.sum(jnp.where(colid == e, comb, 0.0), axis=1,
                    keepdims=True)
        aj = jnp.concatenate(a_parts[j * SPLIT:(j + 1) * SPLIT], axis=1)
        aw.append(aj * w)

    for p in range(SPLIT):
        acc = out_ref[:, p * Hp:(p + 1) * Hp]
        for j in range(E_BLK):
            acc = acc + jax.lax.dot_general(
                aw[j], dp_refs[p][j], (((1,), (1,)), ((), ())),
                preferred_element_type=jnp.float32)
        out_ref[:, p * Hp:(p + 1) * Hp] = acc


@functools.partial(jax.jit, static_argnames=())
def kernel(hidden_states, gate_w, gate_proj, up_proj, down_proj):
    B, S, H = hidden_states.shape
    T = B * S
    hs = hidden_states.reshape(T, H)
    E = gate_w.shape[0]
    I = gate_proj.shape[1]
    Ip = I // SPLIT
    Hp = H // SPLIT

    in_specs = [
        pl.BlockSpec((T, H), lambda i: (0, 0)),
        pl.BlockSpec((E, H), lambda i: (0, 0)),
    ]
    for p in range(SPLIT):
        in_specs.append(
            pl.BlockSpec((E_BLK, Ip, H), lambda i, p=p: (i, p, 0)))
    for p in range(SPLIT):
        in_specs.append(
            pl.BlockSpec((E_BLK, Ip, H), lambda i, p=p: (i, p, 0)))
    for p in range(SPLIT):
        in_specs.append(
            pl.BlockSpec((E_BLK, Hp, I), lambda i, p=p: (i, p, 0)))

    final, logits = pl.pallas_call(
        _moe_body,
        grid=(E // E_BLK,),
        in_specs=in_specs,
        out_specs=[
            pl.BlockSpec((T, H), lambda i: (0, 0)),
            pl.BlockSpec((T, E), lambda i: (0, 0)),
        ],
        out_shape=[
            jax.ShapeDtypeStruct((T, H), jnp.float32),
            jax.ShapeDtypeStruct((T, E), jnp.float32),
        ],
        scratch_shapes=[pltpu.VMEM((T, E), jnp.float32)],
        compiler_params=pltpu.CompilerParams(
            dimension_semantics=("arbitrary",),
        ),
    )(hs, gate_w,
      *([gate_proj] * SPLIT), *([up_proj] * SPLIT), *([down_proj] * SPLIT))

    return final.reshape(B, S, H), logits


# speedup vs baseline: 1.2376x; 1.0234x over previous
import functools

import jax
import jax.numpy as jnp
from jax.experimental import pallas as pl
from jax.experimental.pallas import tpu as pltpu

NUM_EXPERTS = 64
TOP_K = 8
E_BLK = 4
SPLIT = 4


def _moe_body(*refs):
    hs_ref, gw_ref = refs[0], refs[1]
    gp_refs = refs[2:2 + SPLIT]
    up_refs = refs[2 + SPLIT:2 + 2 * SPLIT]
    dp_refs = refs[2 + 2 * SPLIT:2 + 3 * SPLIT]
    out_ref, logits_ref, comb_ref = refs[2 + 3 * SPLIT:]

    i = pl.program_id(0)
    hs = hs_ref[...]
    T, H = hs.shape
    Ip = gp_refs[0].shape[1]
    I = Ip * SPLIT
    Hp = dp_refs[0].shape[1]

    @pl.when(i == 0)
    def _router():
        logits = jax.lax.dot_general(
            hs, gw_ref[...], (((1,), (1,)), ((), ())),
            preferred_element_type=jnp.float32)
        logits_ref[...] = logits
        probs = jax.nn.softmax(logits, axis=1)
        E = probs.shape[1]
        colid = jax.lax.broadcasted_iota(jnp.int32, (T, E), 1)
        comb = jnp.zeros_like(probs)
        p = probs
        for _ in range(TOP_K):
            m = jnp.max(p, axis=1, keepdims=True)
            idx = jnp.where(p == m, colid, E)
            sel = colid == jnp.min(idx, axis=1, keepdims=True)
            comb = jnp.where(sel, p, comb)
            p = jnp.where(sel, -1.0, p)
        comb = comb / jnp.sum(comb, axis=1, keepdims=True)
        comb_ref[...] = comb
        out_ref[...] = jnp.zeros_like(out_ref)

    a_parts = [None] * (E_BLK * SPLIT)
    for p in range(SPLIT):
        gp = gp_refs[p][...].reshape(E_BLK * Ip, H)
        up = up_refs[p][...].reshape(E_BLK * Ip, H)
        g = jax.lax.dot_general(hs, gp, (((1,), (1,)), ((), ())),
                                preferred_element_type=jnp.float32)
        u = jax.lax.dot_general(hs, up, (((1,), (1,)), ((), ())),
                                preferred_element_type=jnp.float32)
        a = g * jax.nn.sigmoid(g) * u
        for j in range(E_BLK):
            a_parts[j * SPLIT + p] = a[:, j * Ip:(j + 1) * Ip]

    E = comb_ref.shape[1]
    colid = jax.lax.broadcasted_iota(jnp.int32, (T, E), 1)
    comb = comb_ref[...]
    aw = []
    for j in range(E_BLK):
        e = i * E_BLK + j
        w = jnp.sum(jnp.where(colid == e, comb, 0.0), axis=1,
                    keepdims=True)
        aj = jnp.concatenate(a_parts[j * SPLIT:(j + 1) * SPLIT], axis=1)
        aw.append(aj * w)

    for p in range(SPLIT):
        acc = out_ref[:, p * Hp:(p + 1) * Hp]
        for j in range(E_BLK):
            acc = acc + jax.lax.dot_general(
                aw[j], dp_refs[p][j], (((1,), (1,)), ((), ())),
                preferred_element_type=jnp.float32)
        out_ref[:, p * Hp:(p + 1) * Hp] = acc


@functools.partial(jax.jit, static_argnames=())
def kernel(hidden_states, gate_w, gate_proj, up_proj, down_proj):
    B, S, H = hidden_states.shape
    T = B * S
    hs = hidden_states.reshape(T, H)
    E = gate_w.shape[0]
    I = gate_proj.shape[1]
    Ip = I // SPLIT
    Hp = H // SPLIT

    in_specs = [
        pl.BlockSpec((T, H), lambda i: (0, 0)),
        pl.BlockSpec((E, H), lambda i: (0, 0)),
    ]
    for p in range(SPLIT):
        in_specs.append(
            pl.BlockSpec((E_BLK, Ip, H), lambda i, p=p: (i, p, 0)))
    for p in range(SPLIT):
        in_specs.append(
            pl.BlockSpec((E_BLK, Ip, H), lambda i, p=p: (i, p, 0)))
    for p in range(SPLIT):
        in_specs.append(
            pl.BlockSpec((E_BLK, Hp, I), lambda i, p=p: (i, p, 0)))

    final, logits = pl.pallas_call(
        _moe_body,
        grid=(E // E_BLK,),
        in_specs=in_specs,
        out_specs=[
            pl.BlockSpec((T, H), lambda i: (0, 0)),
            pl.BlockSpec((T, E), lambda i: (0, 0)),
        ],
        out_shape=[
            jax.ShapeDtypeStruct((T, H), jnp.float32),
            jax.ShapeDtypeStruct((T, E), jnp.float32),
        ],
        scratch_shapes=[pltpu.VMEM((T, E), jnp.float32)],
        compiler_params=pltpu.CompilerParams(
            dimension_semantics=("arbitrary",),
        ),
    )(hs, gate_w,
      *([gate_proj] * SPLIT), *([up_proj] * SPLIT), *([down_proj] * SPLIT))

    return final.reshape(B, S, H), logits


# speedup vs baseline: 1.2746x; 1.0299x over previous
import functools

import jax
import jax.numpy as jnp
from jax import lax
from jax.experimental import pallas as pl
from jax.experimental.pallas import tpu as pltpu

NUM_EXPERTS = 64
TOP_K = 8
E_BLK = 2
NBUF = 3


def _moe_body(hs_ref, gw_ref, gp_hbm, up_hbm, dp_hbm, out_ref, logits_ref,
              gp_buf, up_buf, dp_buf, comb_ref, sems):
    T, H = hs_ref.shape
    E = gw_ref.shape[0]
    I = gp_hbm.shape[1]
    n_steps = E // E_BLK

    def copies(e, slot):
        return (
            pltpu.make_async_copy(
                gp_hbm.at[pl.ds(e * E_BLK, E_BLK)], gp_buf.at[slot],
                sems.at[slot, 0]),
            pltpu.make_async_copy(
                up_hbm.at[pl.ds(e * E_BLK, E_BLK)], up_buf.at[slot],
                sems.at[slot, 1]),
            pltpu.make_async_copy(
                dp_hbm.at[pl.ds(e * E_BLK, E_BLK)], dp_buf.at[slot],
                sems.at[slot, 2]),
        )

    for s in range(NBUF):
        for c in copies(s, s):
            c.start()

    hs = hs_ref[...]
    logits = jax.lax.dot_general(
        hs, gw_ref[...], (((1,), (1,)), ((), ())),
        preferred_element_type=jnp.float32)
    logits_ref[...] = logits
    probs = jax.nn.softmax(logits, axis=1)
    colid = jax.lax.broadcasted_iota(jnp.int32, (T, E), 1)
    comb = jnp.zeros_like(probs)
    p = probs
    for _ in range(TOP_K):
        m = jnp.max(p, axis=1, keepdims=True)
        idx = jnp.where(p == m, colid, E)
        sel = colid == jnp.min(idx, axis=1, keepdims=True)
        comb = jnp.where(sel, p, comb)
        p = jnp.where(sel, -1.0, p)
    comb = comb / jnp.sum(comb, axis=1, keepdims=True)
    comb_ref[...] = comb
    out_ref[...] = jnp.zeros_like(out_ref)

    def step(e, carry):
        slot = lax.rem(e, NBUF)
        for c in copies(e, slot):
            c.wait()

        gp = gp_buf[slot].reshape(E_BLK * I, H)
        up = up_buf[slot].reshape(E_BLK * I, H)
        g = jax.lax.dot_general(hs, gp, (((1,), (1,)), ((), ())),
                                preferred_element_type=jnp.float32)
        u = jax.lax.dot_general(hs, up, (((1,), (1,)), ((), ())),
                                preferred_element_type=jnp.float32)
        a = g * jax.nn.sigmoid(g) * u

        cmb = comb_ref[...]
        acc = out_ref[...]
        for j in range(E_BLK):
            ej = e * E_BLK + j
            w = jnp.sum(jnp.where(colid == ej, cmb, 0.0), axis=1,
                        keepdims=True)
            aw = a[:, j * I:(j + 1) * I] * w
            acc = acc + jax.lax.dot_general(
                aw, dp_buf[slot, j], (((1,), (1,)), ((), ())),
                preferred_element_type=jnp.float32)
        out_ref[...] = acc

        @pl.when(e + NBUF < n_steps)
        def _():
            for c in copies(e + NBUF, slot):
                c.start()

        return carry

    lax.fori_loop(0, n_steps, step, 0)


@functools.partial(jax.jit, static_argnames=())
def kernel(hidden_states, gate_w, gate_proj, up_proj, down_proj):
    B, S, H = hidden_states.shape
    T = B * S
    hs = hidden_states.reshape(T, H)
    E = gate_w.shape[0]
    I = gate_proj.shape[1]

    final, logits = pl.pallas_call(
        _moe_body,
        in_specs=[
            pl.BlockSpec(memory_space=pltpu.VMEM),
            pl.BlockSpec(memory_space=pltpu.VMEM),
            pl.BlockSpec(memory_space=pl.ANY),
            pl.BlockSpec(memory_space=pl.ANY),
            pl.BlockSpec(memory_space=pl.ANY),
        ],
        out_specs=[
            pl.BlockSpec(memory_space=pltpu.VMEM),
            pl.BlockSpec(memory_space=pltpu.VMEM),
        ],
        out_shape=[
            jax.ShapeDtypeStruct((T, H), jnp.float32),
            jax.ShapeDtypeStruct((T, E), jnp.float32),
        ],
        scratch_shapes=[
            pltpu.VMEM((NBUF, E_BLK, I, H), jnp.float32),
            pltpu.VMEM((NBUF, E_BLK, I, H), jnp.float32),
            pltpu.VMEM((NBUF, E_BLK, H, I), jnp.float32),
            pltpu.VMEM((T, E), jnp.float32),
            pltpu.SemaphoreType.DMA((NBUF, 3)),
        ],
    )(hs, gate_w, gate_proj, up_proj, down_proj)

    return final.reshape(B, S, H), logits
